# baseline (speedup 1.0000x reference)
import functools

import jax
import jax.numpy as jnp
from jax import lax
from jax.experimental import pallas as pl
from jax.experimental.pallas import tpu as pltpu
from jax.experimental.pallas import tpu_sc as plsc

_EMBED_DIM = 256
_K = 8192
_N = 8192
_BETA = 0.25

_BR = 1024
_BK = 1024

_NC = 2
_NS = 16
_NW = _NC * _NS
_CHUNK = 128
_ROWS_PER_W = _N // _NW


def _argmin_body(x_ref, e_ref, idx_ref, loss_ref,
                 e2_ref, enorm_ref, fiota_ref, bv_ref, bi_ref):
    ri = pl.program_id(0)
    c = pl.program_id(1)
    nc = pl.num_programs(1)

    @pl.when(ri == 0)
    def _prep_chunk():
        e = e_ref[...]
        e2_ref[:, pl.ds(c * _BK, _BK)] = e * -2.0
        enorm_ref[:, pl.ds(c * _BK, _BK)] = jnp.sum(e * e, axis=0,
                                                    keepdims=True)

    @pl.when(jnp.logical_and(ri == 0, c == 0))
    def _prep_once():
        fiota_ref[...] = lax.broadcasted_iota(
            jnp.int32, (_BR, _BK), 1).astype(jnp.float32)
        loss_ref[...] = jnp.zeros_like(loss_ref)

    f = x_ref[...]
    fnorm = jnp.sum(f * f, axis=1, keepdims=True)
    e2 = e2_ref[:, pl.ds(c * _BK, _BK)]
    sim2 = lax.dot_general(f, e2, (((1,), (0,)), ((), ())),
                           preferred_element_type=jnp.float32)
    d = (fnorm + enorm_ref[:, pl.ds(c * _BK, _BK)]) + sim2
    rowmin = jnp.min(d, axis=1, keepdims=True)
    rowarg = jnp.min(jnp.where(d == rowmin, fiota_ref[...], jnp.float32(1e9)),
                     axis=1, keepdims=True) + c.astype(jnp.float32) * _BK

    @pl.when(c == 0)
    def _first():
        bv_ref[...] = rowmin
        bi_ref[...] = rowarg

    @pl.when(c > 0)
    def _update():
        better = rowmin < bv_ref[...]
        bi_ref[...] = jnp.where(better, rowarg, bi_ref[...])
        bv_ref[...] = jnp.where(better, rowmin, bv_ref[...])

    @pl.when(c == nc - 1)
    def _emit():
        idx_ref[...] = bi_ref[...].astype(jnp.int32).T.reshape(1, 1, _BR)
        loss_ref[...] += jnp.sum(bv_ref[...])[None, None]


def _argmin_call(flat, emb):
    nr = flat.shape[0] // _BR
    return pl.pallas_call(
        _argmin_body,
        grid=(nr, _K // _BK),
        in_specs=[
            pl.BlockSpec((_BR, _EMBED_DIM), lambda ri, c: (ri, 0)),
            pl.BlockSpec((_EMBED_DIM, _BK), lambda ri, c: (0, c)),
        ],
        out_specs=[
            pl.BlockSpec((1, 1, _BR), lambda ri, c: (ri, 0, 0)),
            pl.BlockSpec((1, 1), lambda ri, c: (0, 0)),
        ],
        out_shape=[
            jax.ShapeDtypeStruct((flat.shape[0] // _BR, 1, _BR), jnp.int32),
            jax.ShapeDtypeStruct((1, 1), jnp.float32),
        ],
        scratch_shapes=[
            pltpu.VMEM((_EMBED_DIM, _K), jnp.float32),
            pltpu.VMEM((1, _K), jnp.float32),
            pltpu.VMEM((_BR, _BK), jnp.float32),
            pltpu.VMEM((_BR, 1), jnp.float32),
            pltpu.VMEM((_BR, 1), jnp.float32),
        ],
    )(flat, emb)


def _gather_body(table_hbm, idx_hbm, out_hbm, idx_v, rows_v, sem):
    wid = lax.axis_index("s") * _NC + lax.axis_index("c")
    pltpu.sync_copy(idx_hbm.at[wid], idx_v)
    for j in range(2):
        pltpu.async_copy(table_hbm.at[idx_v.at[j]], rows_v.at[j], sem).wait()
    for j in range(2):
        pltpu.sync_copy(
            rows_v.at[j],
            out_hbm.at[pl.ds(wid * _ROWS_PER_W + j * _CHUNK, _CHUNK)])


def _gather_call(table, idx3):
    mesh = plsc.VectorSubcoreMesh(core_axis_name="c", subcore_axis_name="s")
    gk = functools.partial(
        pl.kernel,
        out_type=jax.ShapeDtypeStruct((_N, _EMBED_DIM), jnp.float32),
        mesh=mesh,
        scratch_types=[
            pltpu.VMEM((2, _CHUNK), jnp.int32),
            pltpu.VMEM((2, _CHUNK, _EMBED_DIM), jnp.float32),
            pltpu.SemaphoreType.DMA,
        ],
    )(_gather_body)
    return gk(table, idx3)


def kernel(x, embedding):
    shape = x.shape
    flat = x.reshape(-1, _EMBED_DIM)
    idx2, loss_acc = _argmin_call(flat, embedding)
    idx3 = idx2.reshape(_NW, 2, _CHUNK)
    q = _gather_call(embedding.T, idx3)
    m = loss_acc[0, 0] / jnp.float32(_N * _EMBED_DIM)
    loss = _BETA * m + m
    return q.reshape(shape), loss


# speedup vs baseline: 1.3135x; 1.3135x over previous
import functools

import jax
import jax.numpy as jnp
from jax import lax
from jax.experimental import pallas as pl
from jax.experimental.pallas import tpu as pltpu
from jax.experimental.pallas import tpu_sc as plsc

_EMBED_DIM = 256
_K = 8192
_N = 8192
_BETA = 0.25

_BR = 1024
_BK = 1024

_NC = 2
_NS = 16
_NW = _NC * _NS
_CHUNK = 128
_ROWS_PER_W = _N // _NW


def _argmin_body(x_ref, e_ref, idx_ref, loss_ref, e2_ref, enorm_ref):
    ri = pl.program_id(0)

    @pl.when(ri == 0)
    def _prep():
        e = e_ref[...]
        e2_ref[...] = e * -2.0
        enorm_ref[...] = jnp.sum(e * e, axis=0, keepdims=True)
        loss_ref[...] = jnp.zeros_like(loss_ref)

    f = x_ref[...]
    fnorm = jnp.sum(f * f, axis=1, keepdims=True)
    fiota = lax.broadcasted_iota(jnp.int32, (_BR, _BK), 1).astype(jnp.float32)
    best_val = None
    best_idx = None
    for c in range(_K // _BK):
        e2 = e2_ref[:, c * _BK:(c + 1) * _BK]
        sim2 = lax.dot_general(f, e2, (((1,), (0,)), ((), ())),
                               preferred_element_type=jnp.float32)
        d = (fnorm + enorm_ref[:, c * _BK:(c + 1) * _BK]) + sim2
        rowmin = jnp.min(d, axis=1, keepdims=True)
        rowarg = jnp.min(jnp.where(d == rowmin, fiota, jnp.float32(1e9)),
                         axis=1, keepdims=True) + jnp.float32(c * _BK)
        if c == 0:
            best_val, best_idx = rowmin, rowarg
        else:
            better = rowmin < best_val
            best_val = jnp.where(better, rowmin, best_val)
            best_idx = jnp.where(better, rowarg, best_idx)
    idx_ref[...] = best_idx.astype(jnp.int32).T.reshape(1, 1, _BR)
    loss_ref[...] += jnp.sum(best_val)[None, None]


def _argmin_call(flat, emb):
    nr = flat.shape[0] // _BR
    return pl.pallas_call(
        _argmin_body,
        grid=(nr,),
        in_specs=[
            pl.BlockSpec((_BR, _EMBED_DIM), lambda ri: (ri, 0)),
            pl.BlockSpec((_EMBED_DIM, _K), lambda ri: (0, 0)),
        ],
        out_specs=[
            pl.BlockSpec((1, 1, _BR), lambda ri: (ri, 0, 0)),
            pl.BlockSpec((1, 1), lambda ri: (0, 0)),
        ],
        out_shape=[
            jax.ShapeDtypeStruct((flat.shape[0] // _BR, 1, _BR), jnp.int32),
            jax.ShapeDtypeStruct((1, 1), jnp.float32),
        ],
        scratch_shapes=[
            pltpu.VMEM((_EMBED_DIM, _K), jnp.float32),
            pltpu.VMEM((1, _K), jnp.float32),
        ],
    )(flat, emb)


def _gather_body(table_hbm, idx_hbm, out_hbm, idx_v, rows_v, sem):
    wid = lax.axis_index("s") * _NC + lax.axis_index("c")
    pltpu.sync_copy(idx_hbm.at[wid], idx_v)
    for j in range(2):
        pltpu.async_copy(table_hbm.at[idx_v.at[j]], rows_v.at[j], sem).wait()
    for j in range(2):
        pltpu.sync_copy(
            rows_v.at[j],
            out_hbm.at[pl.ds(wid * _ROWS_PER_W + j * _CHUNK, _CHUNK)])


def _gather_call(table, idx3):
    mesh = plsc.VectorSubcoreMesh(core_axis_name="c", subcore_axis_name="s")
    gk = functools.partial(
        pl.kernel,
        out_type=jax.ShapeDtypeStruct((_N, _EMBED_DIM), jnp.float32),
        mesh=mesh,
        scratch_types=[
            pltpu.VMEM((2, _CHUNK), jnp.int32),
            pltpu.VMEM((2, _CHUNK, _EMBED_DIM), jnp.float32),
            pltpu.SemaphoreType.DMA,
        ],
    )(_gather_body)
    return gk(table, idx3)


def kernel(x, embedding):
    shape = x.shape
    flat = x.reshape(-1, _EMBED_DIM)
    idx2, loss_acc = _argmin_call(flat, embedding)
    idx3 = idx2.reshape(_NW, 2, _CHUNK)
    q = _gather_call(embedding.T, idx3)
    m = loss_acc[0, 0] / jnp.float32(_N * _EMBED_DIM)
    loss = _BETA * m + m
    return q.reshape(shape), loss


# speedup vs baseline: 1.3230x; 1.0073x over previous
import functools

import jax
import jax.numpy as jnp
from jax import lax
from jax.experimental import pallas as pl
from jax.experimental.pallas import tpu as pltpu
from jax.experimental.pallas import tpu_sc as plsc

_EMBED_DIM = 256
_K = 8192
_N = 8192
_BETA = 0.25

_BR = 1024
_BK = 1024

_NC = 2
_NS = 16
_NW = _NC * _NS
_CHUNK = 128
_ROWS_PER_W = _N // _NW


def _argmin_body(x_ref, e_ref, idx_ref, loss_ref, e2_ref, enorm_ref):
    ri = pl.program_id(0)

    @pl.when(ri == 0)
    def _prep():
        e = e_ref[...]
        e2_ref[...] = e * -2.0
        enorm_ref[...] = jnp.sum(e * e, axis=0, keepdims=True)
        loss_ref[...] = jnp.zeros_like(loss_ref)

    f = x_ref[...]
    fnorm = jnp.sum(f * f, axis=1, keepdims=True)
    fiota = lax.broadcasted_iota(jnp.int32, (_BR, _BK), 1).astype(jnp.float32)
    best_val = None
    best_idx = None
    for c in range(_K // _BK):
        e2 = e2_ref[:, c * _BK:(c + 1) * _BK]
        sim2 = lax.dot_general(f, e2, (((1,), (0,)), ((), ())),
                               preferred_element_type=jnp.float32)
        d = (fnorm + enorm_ref[:, c * _BK:(c + 1) * _BK]) + sim2
        rowmin = jnp.min(d, axis=1, keepdims=True)
        rowarg = jnp.min(jnp.where(d == rowmin, fiota, jnp.float32(1e9)),
                         axis=1, keepdims=True) + jnp.float32(c * _BK)
        if c == 0:
            best_val, best_idx = rowmin, rowarg
        else:
            better = rowmin < best_val
            best_val = jnp.where(better, rowmin, best_val)
            best_idx = jnp.where(better, rowarg, best_idx)
    idx_ref[...] = best_idx.astype(jnp.int32).T.reshape(1, 1, _BR)
    loss_ref[...] += jnp.sum(best_val)[None, None]

    @pl.when(ri == pl.num_programs(0) - 1)
    def _finalize():
        m = loss_ref[0, 0] / jnp.float32(_N * _EMBED_DIM)
        loss_ref[...] = (_BETA * m + m)[None, None]


def _argmin_call(flat, emb):
    nr = flat.shape[0] // _BR
    return pl.pallas_call(
        _argmin_body,
        grid=(nr,),
        in_specs=[
            pl.BlockSpec((_BR, _EMBED_DIM), lambda ri: (ri, 0)),
            pl.BlockSpec((_EMBED_DIM, _K), lambda ri: (0, 0)),
        ],
        out_specs=[
            pl.BlockSpec((1, 1, _BR), lambda ri: (ri, 0, 0)),
            pl.BlockSpec((1, 1), lambda ri: (0, 0)),
        ],
        out_shape=[
            jax.ShapeDtypeStruct((flat.shape[0] // _BR, 1, _BR), jnp.int32),
            jax.ShapeDtypeStruct((1, 1), jnp.float32),
        ],
        scratch_shapes=[
            pltpu.VMEM((_EMBED_DIM, _K), jnp.float32),
            pltpu.VMEM((1, _K), jnp.float32),
        ],
    )(flat, emb)


def _gather_body(table_hbm, idx_hbm, out_hbm, idx_v, rows_v, sem):
    wid = lax.axis_index("s") * _NC + lax.axis_index("c")
    pltpu.sync_copy(idx_hbm.at[wid], idx_v)
    for j in range(2):
        pltpu.async_copy(table_hbm.at[idx_v.at[j]], rows_v.at[j], sem).wait()
    for j in range(2):
        pltpu.sync_copy(
            rows_v.at[j],
            out_hbm.at[pl.ds(wid * _ROWS_PER_W + j * _CHUNK, _CHUNK)])


def _gather_call(table, idx3):
    mesh = plsc.VectorSubcoreMesh(core_axis_name="c", subcore_axis_name="s")
    gk = functools.partial(
        pl.kernel,
        out_type=jax.ShapeDtypeStruct((_N, _EMBED_DIM), jnp.float32),
        mesh=mesh,
        scratch_types=[
            pltpu.VMEM((2, _CHUNK), jnp.int32),
            pltpu.VMEM((2, _CHUNK, _EMBED_DIM), jnp.float32),
            pltpu.SemaphoreType.DMA,
        ],
    )(_gather_body)
    return gk(table, idx3)


def kernel(x, embedding):
    shape = x.shape
    flat = x.reshape(-1, _EMBED_DIM)
    idx2, loss_acc = _argmin_call(flat, embedding)
    idx3 = idx2.reshape(_NW, 2, _CHUNK)
    q = _gather_call(embedding.T, idx3)
    return q.reshape(shape), loss_acc.reshape(())
